# Initial kernel scaffold; baseline (speedup 1.0000x reference)
import jax
import jax.numpy as jnp
from jax.experimental import pallas as pl


def kernel(x, edge_index, batch, Wl1, Wr1, b1, Wl2, Wr2, b2, Wl3, Wr3, b3, Wl4, Wr4, b4, Wl5, Wr5, b5, gn_w1, gn_b1, gn_ms1, gn_w2, gn_b2, gn_ms2, gn_w3, gn_b3, gn_ms3, gn_w4, gn_b4, gn_ms4):
    raise NotImplementedError("write your pallas kernel here")


# speedup vs baseline: 5.0528x; 5.0528x over previous
import functools

import jax
import jax.numpy as jnp
from jax import lax
from jax.experimental import pallas as pl
from jax.experimental.pallas import tpu as pltpu
from jax.experimental.pallas import tpu_sc as plsc

N = 10000
E = 320000
IN_F = 128
HID = 512
OUT_F = 121
G = 16

NPAD = 10240
ROWS_PER_TILE = NPAD // 16
CHUNK = 128
EB = 128

IB32 = 40
NIB32 = 2
K32 = IB32 * NIB32
IB16 = 32
NIB16 = 5
K16 = IB16 * NIB16


def _pad_edges(src, dst, nshard, k):
    tot = nshard * k * EB
    pad = tot - E
    ar = jnp.arange(pad, dtype=jnp.int32)
    src_p = jnp.concatenate([src, (ar * 61) % N]).reshape(nshard, k, EB)
    dst_p = jnp.concatenate([dst, N + (ar % (NPAD - N))]).reshape(nshard, k, EB)
    return src_p, dst_p


def _sc_agg1(x_hbm, src_hbm, dst_hbm, z2_hbm, z1_hbm,
             s_out, deg_out,
             idx_s, idx_d, rows, ones_v, acc, dacc, sem):
    c = lax.axis_index("c")
    s = lax.axis_index("s")
    wid = s * 2 + c
    r0 = s * ROWS_PER_TILE
    pltpu.sync_copy(z2_hbm.at[pl.ds(r0, ROWS_PER_TILE)],
                    acc.at[pl.ds(r0, ROWS_PER_TILE)])
    pltpu.sync_copy(z1_hbm.at[pl.ds(r0, ROWS_PER_TILE)],
                    dacc.at[pl.ds(r0, ROWS_PER_TILE)])
    for j in range(8):
        ones_v[pl.ds(j * 16, 16)] = jnp.ones((16,), jnp.float32)
    plsc.subcore_barrier()

    def blk(ib, carry):
        pltpu.sync_copy(src_hbm.at[wid].at[pl.ds(ib * IB32, IB32)], idx_s)
        pltpu.sync_copy(dst_hbm.at[wid].at[pl.ds(ib * IB32, IB32)], idx_d)

        def body(j, carry2):
            pltpu.async_copy(x_hbm.at[idx_s.at[j]], rows, sem).wait()
            pltpu.sync_copy(rows, acc.at[idx_d.at[j]], add=True)
            pltpu.sync_copy(ones_v, dacc.at[idx_d.at[j]], add=True)
            return carry2

        return lax.fori_loop(0, IB32, body, carry)

    lax.fori_loop(0, NIB32, blk, 0)
    plsc.subcore_barrier()
    pltpu.sync_copy(acc.at[pl.ds(r0, ROWS_PER_TILE)],
                    s_out.at[c].at[pl.ds(r0, ROWS_PER_TILE)])
    pltpu.sync_copy(dacc.at[pl.ds(r0, ROWS_PER_TILE)],
                    deg_out.at[c].at[pl.ds(r0, ROWS_PER_TILE)])


def _sc_agg4(h0, h1, h2, h3, src_hbm, dst_hbm, z2_hbm,
             s_out,
             idx_s, idx_d, rows, acc, sem):
    c = lax.axis_index("c")
    s = lax.axis_index("s")
    r0 = s * ROWS_PER_TILE

    def do_chunk(h_hbm, out_idx):
        pltpu.sync_copy(z2_hbm.at[pl.ds(r0, ROWS_PER_TILE)],
                        acc.at[pl.ds(r0, ROWS_PER_TILE)])
        plsc.subcore_barrier()

        def blk(ib, carry):
            pltpu.sync_copy(src_hbm.at[s].at[pl.ds(ib * IB16, IB16)], idx_s)
            pltpu.sync_copy(dst_hbm.at[s].at[pl.ds(ib * IB16, IB16)], idx_d)

            def body(j, carry2):
                pltpu.async_copy(h_hbm.at[idx_s.at[j]], rows, sem).wait()
                pltpu.sync_copy(rows, acc.at[idx_d.at[j]], add=True)
                return carry2

            return lax.fori_loop(0, IB16, body, carry)

        lax.fori_loop(0, NIB16, blk, 0)
        plsc.subcore_barrier()
        pltpu.sync_copy(acc.at[pl.ds(r0, ROWS_PER_TILE)],
                        s_out.at[out_idx].at[pl.ds(r0, ROWS_PER_TILE)])
        plsc.subcore_barrier()

    @pl.when(c == 0)
    def _():
        do_chunk(h0, 0)
        do_chunk(h1, 1)

    @pl.when(c == 1)
    def _():
        do_chunk(h2, 2)
        do_chunk(h3, 3)


_MESH = plsc.VectorSubcoreMesh(core_axis_name="c", subcore_axis_name="s")

_agg1_call = pl.kernel(
    _sc_agg1,
    out_type=[jax.ShapeDtypeStruct((2, NPAD, CHUNK), jnp.float32),
              jax.ShapeDtypeStruct((2, NPAD), jnp.float32)],
    mesh=_MESH,
    scratch_types=[
        pltpu.VMEM((IB32, EB), jnp.int32),
        pltpu.VMEM((IB32, EB), jnp.int32),
        pltpu.VMEM((EB, CHUNK), jnp.float32),
        pltpu.VMEM((EB,), jnp.float32),
        pltpu.VMEM_SHARED((NPAD, CHUNK), jnp.float32),
        pltpu.VMEM_SHARED((NPAD,), jnp.float32),
        pltpu.SemaphoreType.DMA,
    ],
)

_agg4_call = pl.kernel(
    _sc_agg4,
    out_type=jax.ShapeDtypeStruct((4, NPAD, CHUNK), jnp.float32),
    mesh=_MESH,
    scratch_types=[
        pltpu.VMEM((IB16, EB), jnp.int32),
        pltpu.VMEM((IB16, EB), jnp.int32),
        pltpu.VMEM((EB, CHUNK), jnp.float32),
        pltpu.VMEM_SHARED((NPAD, CHUNK), jnp.float32),
        pltpu.SemaphoreType.DMA,
    ],
)


B = 1000
NB = N // B


def _onehot_gb(batch_row):
    return (lax.broadcasted_iota(jnp.int32, (G, B), 0)
            == batch_row[None, :]).astype(jnp.float32)


def _p1l1_body(s_ref, deg_ref, x_ref, batch_ref, wl_ref, wr_ref, b_ref,
               pre_ref, stats_ref, gs, gq, gc):
    i = pl.program_id(0)
    invd = 1.0 / jnp.maximum(deg_ref[:, 0], 1.0)
    srow = (s_ref[0] + s_ref[1]) * invd[:, None]
    pre = (jnp.dot(srow, wl_ref[...], preferred_element_type=jnp.float32)
           + jnp.dot(x_ref[...], wr_ref[...], preferred_element_type=jnp.float32)
           + b_ref[...])
    pre_ref[...] = pre
    oh = _onehot_gb(batch_ref[0, 0, :])

    @pl.when(i == 0)
    def _():
        gs[...] = jnp.zeros((G, HID), jnp.float32)
        gq[...] = jnp.zeros((G, HID), jnp.float32)
        gc[...] = jnp.zeros((G, HID), jnp.float32)

    gs[...] += jnp.dot(oh, pre, preferred_element_type=jnp.float32)
    gq[...] += jnp.dot(oh, pre * pre, preferred_element_type=jnp.float32)
    gc[...] += jnp.broadcast_to(jnp.sum(oh, axis=1, keepdims=True), (G, HID))

    @pl.when(i == NB - 1)
    def _():
        stats_ref[0] = gs[...]
        stats_ref[1] = gq[...]
        stats_ref[2] = gc[...]


def _p1_body(s_ref, deg_ref, h_ref, batch_ref, wl_ref, wr_ref, b_ref,
             pre_ref, stats_ref, gs, gq, gc):
    i = pl.program_id(0)
    invd = 1.0 / jnp.maximum(deg_ref[:, 0], 1.0)
    acc = jnp.zeros((B, HID), jnp.float32)
    for c in range(4):
        sl = pl.ds(c * CHUNK, CHUNK)
        acc = acc + jnp.dot(s_ref[c] * invd[:, None], wl_ref[sl, :],
                            preferred_element_type=jnp.float32)
        acc = acc + jnp.dot(h_ref[c], wr_ref[sl, :],
                            preferred_element_type=jnp.float32)
    pre = acc + b_ref[...]
    pre_ref[...] = pre
    oh = _onehot_gb(batch_ref[0, 0, :])

    @pl.when(i == 0)
    def _():
        gs[...] = jnp.zeros((G, HID), jnp.float32)
        gq[...] = jnp.zeros((G, HID), jnp.float32)
        gc[...] = jnp.zeros((G, HID), jnp.float32)

    gs[...] += jnp.dot(oh, pre, preferred_element_type=jnp.float32)
    gq[...] += jnp.dot(oh, pre * pre, preferred_element_type=jnp.float32)
    gc[...] += jnp.broadcast_to(jnp.sum(oh, axis=1, keepdims=True), (G, HID))

    @pl.when(i == NB - 1)
    def _():
        stats_ref[0] = gs[...]
        stats_ref[1] = gq[...]
        stats_ref[2] = gc[...]


def _p2_body(pre_ref, batch_ref, stats_ref, w_ref, b_ref, ms_ref, out_ref):
    cnt = jnp.maximum(stats_ref[2], 1.0)
    mean = stats_ref[0] / cnt
    msv = ms_ref[...]
    var = stats_ref[1] / cnt - mean * mean * (2.0 * msv - msv * msv)
    rstd = lax.rsqrt(var + 1e-5)
    bt = batch_ref[0, 0, :]
    oht = (bt[:, None]
           == lax.broadcasted_iota(jnp.int32, (B, G), 1)).astype(jnp.float32)
    rmean = jnp.dot(oht, mean, preferred_element_type=jnp.float32)
    rrstd = jnp.dot(oht, rstd, preferred_element_type=jnp.float32)
    pre = pre_ref[...]
    val = (pre - rmean * msv) * rrstd * w_ref[...] + b_ref[...]
    val = jnp.maximum(val, 0.0)
    for c in range(4):
        out_ref[c] = val[:, c * CHUNK:(c + 1) * CHUNK]


def _p5_body(s_ref, deg_ref, h_ref, wl_ref, wr_ref, b_ref, out_ref):
    invd = 1.0 / jnp.maximum(deg_ref[:, 0], 1.0)
    acc = jnp.zeros((B, 128), jnp.float32)
    for c in range(4):
        sl = pl.ds(c * CHUNK, CHUNK)
        acc = acc + jnp.dot(s_ref[c] * invd[:, None], wl_ref[sl, :],
                            preferred_element_type=jnp.float32)
        acc = acc + jnp.dot(h_ref[c], wr_ref[sl, :],
                            preferred_element_type=jnp.float32)
    out_ref[...] = acc + b_ref[...]


def _p1l1_call(s2, degc, x, batch3, wlT, wrT, b2):
    return pl.pallas_call(
        _p1l1_body,
        grid=(NB,),
        in_specs=[
            pl.BlockSpec((2, B, CHUNK), lambda i: (0, i, 0)),
            pl.BlockSpec((B, 1), lambda i: (i, 0)),
            pl.BlockSpec((B, IN_F), lambda i: (i, 0)),
            pl.BlockSpec((1, 1, B), lambda i: (i, 0, 0)),
            pl.BlockSpec((IN_F, HID), lambda i: (0, 0)),
            pl.BlockSpec((IN_F, HID), lambda i: (0, 0)),
            pl.BlockSpec((1, HID), lambda i: (0, 0)),
        ],
        out_specs=[
            pl.BlockSpec((B, HID), lambda i: (i, 0)),
            pl.BlockSpec((3, G, HID), lambda i: (0, 0, 0)),
        ],
        out_shape=[jax.ShapeDtypeStruct((N, HID), jnp.float32),
                   jax.ShapeDtypeStruct((3, G, HID), jnp.float32)],
        scratch_shapes=[pltpu.VMEM((G, HID), jnp.float32)] * 3,
    )(s2, degc, x, batch3, wlT, wrT, b2)


def _p1_call(s4, degc, hc, batch3, wlT, wrT, b2):
    return pl.pallas_call(
        _p1_body,
        grid=(NB,),
        in_specs=[
            pl.BlockSpec((4, B, CHUNK), lambda i: (0, i, 0)),
            pl.BlockSpec((B, 1), lambda i: (i, 0)),
            pl.BlockSpec((4, B, CHUNK), lambda i: (0, i, 0)),
            pl.BlockSpec((1, 1, B), lambda i: (i, 0, 0)),
            pl.BlockSpec((HID, HID), lambda i: (0, 0)),
            pl.BlockSpec((HID, HID), lambda i: (0, 0)),
            pl.BlockSpec((1, HID), lambda i: (0, 0)),
        ],
        out_specs=[
            pl.BlockSpec((B, HID), lambda i: (i, 0)),
            pl.BlockSpec((3, G, HID), lambda i: (0, 0, 0)),
        ],
        out_shape=[jax.ShapeDtypeStruct((N, HID), jnp.float32),
                   jax.ShapeDtypeStruct((3, G, HID), jnp.float32)],
        scratch_shapes=[pltpu.VMEM((G, HID), jnp.float32)] * 3,
    )(s4, degc, hc, batch3, wlT, wrT, b2)


def _p2_call(pre, batch3, stats, w2, b2, ms2):
    return pl.pallas_call(
        _p2_body,
        grid=(NB,),
        in_specs=[
            pl.BlockSpec((B, HID), lambda i: (i, 0)),
            pl.BlockSpec((1, 1, B), lambda i: (i, 0, 0)),
            pl.BlockSpec((3, G, HID), lambda i: (0, 0, 0)),
            pl.BlockSpec((1, HID), lambda i: (0, 0)),
            pl.BlockSpec((1, HID), lambda i: (0, 0)),
            pl.BlockSpec((1, HID), lambda i: (0, 0)),
        ],
        out_specs=pl.BlockSpec((4, B, CHUNK), lambda i: (0, i, 0)),
        out_shape=jax.ShapeDtypeStruct((4, N, CHUNK), jnp.float32),
    )(pre, batch3, stats, w2, b2, ms2)


def _p5_call(s4, degc, hc, wlT, wrT, b2):
    return pl.pallas_call(
        _p5_body,
        grid=(NB,),
        in_specs=[
            pl.BlockSpec((4, B, CHUNK), lambda i: (0, i, 0)),
            pl.BlockSpec((B, 1), lambda i: (i, 0)),
            pl.BlockSpec((4, B, CHUNK), lambda i: (0, i, 0)),
            pl.BlockSpec((HID, 128), lambda i: (0, 0)),
            pl.BlockSpec((HID, 128), lambda i: (0, 0)),
            pl.BlockSpec((1, 128), lambda i: (0, 0)),
        ],
        out_specs=pl.BlockSpec((B, 128), lambda i: (i, 0)),
        out_shape=jax.ShapeDtypeStruct((N, 128), jnp.float32),
    )(s4, degc, hc, wlT, wrT, b2)


def kernel(x, edge_index, batch, Wl1, Wr1, b1, Wl2, Wr2, b2, Wl3, Wr3, b3,
           Wl4, Wr4, b4, Wl5, Wr5, b5, gn_w1, gn_b1, gn_ms1, gn_w2, gn_b2,
           gn_ms2, gn_w3, gn_b3, gn_ms3, gn_w4, gn_b4, gn_ms4):
    src, dst = edge_index[0], edge_index[1]
    src1, dst1 = _pad_edges(src, dst, 32, K32)
    src4, dst4 = _pad_edges(src, dst, 16, K16)
    z2 = jnp.zeros((NPAD, CHUNK), jnp.float32)
    z1 = jnp.zeros((NPAD,), jnp.float32)
    batch3 = batch.reshape(NB, 1, B)

    s2, deg = _agg1_call(x, src1, dst1, z2, z1)
    degc = (deg[0] + deg[1]).reshape(NPAD, 1)
    pre, stats = _p1l1_call(s2, degc, x, batch3, Wl1.T, Wr1.T,
                            b1.reshape(1, HID))
    h = _p2_call(pre, batch3, stats, gn_w1.reshape(1, HID),
                 gn_b1.reshape(1, HID), gn_ms1.reshape(1, HID))

    for Wl, Wr, bb, gw, gb, gms in ((Wl2, Wr2, b2, gn_w2, gn_b2, gn_ms2),
                                    (Wl3, Wr3, b3, gn_w3, gn_b3, gn_ms3),
                                    (Wl4, Wr4, b4, gn_w4, gn_b4, gn_ms4)):
        s4 = _agg4_call(h[0], h[1], h[2], h[3], src4, dst4, z2)
        pre, stats = _p1_call(s4, degc, h, batch3, Wl.T, Wr.T,
                              bb.reshape(1, HID))
        h = _p2_call(pre, batch3, stats, gw.reshape(1, HID),
                     gb.reshape(1, HID), gms.reshape(1, HID))

    s4 = _agg4_call(h[0], h[1], h[2], h[3], src4, dst4, z2)
    wl5 = jnp.pad(Wl5.T, ((0, 0), (0, 128 - OUT_F)))
    wr5 = jnp.pad(Wr5.T, ((0, 0), (0, 128 - OUT_F)))
    b5p = jnp.pad(b5, (0, 128 - OUT_F)).reshape(1, 128)
    out = _p5_call(s4, degc, h, wl5, wr5, b5p)
    return out[:, :OUT_F]


# speedup vs baseline: 7.1565x; 1.4163x over previous
import functools

import jax
import jax.numpy as jnp
from jax import lax
from jax.experimental import pallas as pl
from jax.experimental.pallas import tpu as pltpu
from jax.experimental.pallas import tpu_sc as plsc

N = 10000
E = 320000
IN_F = 128
HID = 512
OUT_F = 121
G = 16

NPAD = 10240
ROWS_PER_TILE = NPAD // 16
CHUNK = 128
EB = 128

IB32 = 40
NIB32 = 2
K32 = IB32 * NIB32
IB16 = 32
NIB16 = 5
K16 = IB16 * NIB16


def _pad_edges(src, dst, nshard, k):
    tot = nshard * k * EB
    pad = tot - E
    ar = jnp.arange(pad, dtype=jnp.int32)
    src_p = jnp.concatenate([src, (ar * 61) % N]).reshape(nshard, k, EB)
    dst_p = jnp.concatenate([dst, N + (ar % (NPAD - N))]).reshape(nshard, k, EB)
    return src_p, dst_p


def _edge_sweep(h_hbm, src_hbm, dst_hbm, shard, acc,
                idx_s, idx_d, rows_a, rows_b, gsa, gsb, ssa, ssb,
                ib_n, ib_sz, per_edge=None):
    def blk(ib, carry):
        pltpu.sync_copy(src_hbm.at[shard].at[pl.ds(ib * ib_sz, ib_sz)], idx_s)
        pltpu.sync_copy(dst_hbm.at[shard].at[pl.ds(ib * ib_sz, ib_sz)], idx_d)
        pltpu.async_copy(h_hbm.at[idx_s.at[0]], rows_a, gsa)
        pltpu.async_copy(h_hbm.at[idx_s.at[1]], rows_b, gsb)

        def pair(p, carry2):
            j = 2 * p
            pltpu.make_async_copy(h_hbm.at[idx_s.at[j]], rows_a, gsa).wait()
            sa = pltpu.async_copy(rows_a, acc.at[idx_d.at[j]], ssa, add=True)
            if per_edge is not None:
                per_edge(j)
            pltpu.make_async_copy(h_hbm.at[idx_s.at[j + 1]], rows_b,
                                  gsb).wait()
            sb = pltpu.async_copy(rows_b, acc.at[idx_d.at[j + 1]], ssb,
                                  add=True)
            if per_edge is not None:
                per_edge(j + 1)
            sa.wait()

            @pl.when(p + 1 < ib_sz // 2)
            def _():
                pltpu.async_copy(h_hbm.at[idx_s.at[j + 2]], rows_a, gsa)

            sb.wait()

            @pl.when(p + 1 < ib_sz // 2)
            def _():
                pltpu.async_copy(h_hbm.at[idx_s.at[j + 3]], rows_b, gsb)

            return carry2

        return lax.fori_loop(0, ib_sz // 2, pair, carry)

    lax.fori_loop(0, ib_n, blk, 0)


def _sc_agg1(x_hbm, src_hbm, dst_hbm, z2_hbm, z1_hbm,
             s_out, deg_out,
             idx_s, idx_d, rows_a, rows_b, ones_v, acc, dacc,
             gsa, gsb, ssa, ssb):
    c = lax.axis_index("c")
    s = lax.axis_index("s")
    wid = s * 2 + c
    r0 = s * ROWS_PER_TILE
    pltpu.sync_copy(z2_hbm.at[pl.ds(r0, ROWS_PER_TILE)],
                    acc.at[pl.ds(r0, ROWS_PER_TILE)])
    pltpu.sync_copy(z1_hbm.at[pl.ds(r0, ROWS_PER_TILE)],
                    dacc.at[pl.ds(r0, ROWS_PER_TILE)])
    for j in range(8):
        ones_v[pl.ds(j * 16, 16)] = jnp.ones((16,), jnp.float32)
    plsc.subcore_barrier()

    def deg_edge(j):
        pltpu.sync_copy(ones_v, dacc.at[idx_d.at[j]], add=True)

    _edge_sweep(x_hbm, src_hbm, dst_hbm, wid, acc,
                idx_s, idx_d, rows_a, rows_b, gsa, gsb, ssa, ssb,
                NIB32, IB32, per_edge=deg_edge)
    plsc.subcore_barrier()
    pltpu.sync_copy(acc.at[pl.ds(r0, ROWS_PER_TILE)],
                    s_out.at[c].at[pl.ds(r0, ROWS_PER_TILE)])
    pltpu.sync_copy(dacc.at[pl.ds(r0, ROWS_PER_TILE)],
                    deg_out.at[c].at[pl.ds(r0, ROWS_PER_TILE)])


def _sc_agg1nd(x_hbm, src_hbm, dst_hbm, z2_hbm,
               s_out,
               idx_s, idx_d, rows_a, rows_b, acc, gsa, gsb, ssa, ssb):
    c = lax.axis_index("c")
    s = lax.axis_index("s")
    wid = s * 2 + c
    r0 = s * ROWS_PER_TILE
    pltpu.sync_copy(z2_hbm.at[pl.ds(r0, ROWS_PER_TILE)],
                    acc.at[pl.ds(r0, ROWS_PER_TILE)])
    plsc.subcore_barrier()
    _edge_sweep(x_hbm, src_hbm, dst_hbm, wid, acc,
                idx_s, idx_d, rows_a, rows_b, gsa, gsb, ssa, ssb,
                NIB32, IB32)
    plsc.subcore_barrier()
    pltpu.sync_copy(acc.at[pl.ds(r0, ROWS_PER_TILE)],
                    s_out.at[c].at[pl.ds(r0, ROWS_PER_TILE)])


def _sc_agg4(h0, h1, h2, h3, src_hbm, dst_hbm, z2_hbm,
             s_out,
             idx_s, idx_d, rows_a, rows_b, acc, gsa, gsb, ssa, ssb):
    c = lax.axis_index("c")
    s = lax.axis_index("s")
    r0 = s * ROWS_PER_TILE

    def do_chunk(h_hbm, out_idx):
        pltpu.sync_copy(z2_hbm.at[pl.ds(r0, ROWS_PER_TILE)],
                        acc.at[pl.ds(r0, ROWS_PER_TILE)])
        plsc.subcore_barrier()
        _edge_sweep(h_hbm, src_hbm, dst_hbm, s, acc,
                    idx_s, idx_d, rows_a, rows_b, gsa, gsb, ssa, ssb,
                    NIB16, IB16)
        plsc.subcore_barrier()
        pltpu.sync_copy(acc.at[pl.ds(r0, ROWS_PER_TILE)],
                        s_out.at[out_idx].at[pl.ds(r0, ROWS_PER_TILE)])
        plsc.subcore_barrier()

    @pl.when(c == 0)
    def _():
        do_chunk(h0, 0)
        do_chunk(h1, 1)

    @pl.when(c == 1)
    def _():
        do_chunk(h2, 2)
        do_chunk(h3, 3)


_MESH = plsc.VectorSubcoreMesh(core_axis_name="c", subcore_axis_name="s")

_agg1_call = pl.kernel(
    _sc_agg1,
    out_type=[jax.ShapeDtypeStruct((2, NPAD, CHUNK), jnp.float32),
              jax.ShapeDtypeStruct((2, NPAD), jnp.float32)],
    mesh=_MESH,
    scratch_types=[
        pltpu.VMEM((IB32, EB), jnp.int32),
        pltpu.VMEM((IB32, EB), jnp.int32),
        pltpu.VMEM((EB, CHUNK), jnp.float32),
        pltpu.VMEM((EB, CHUNK), jnp.float32),
        pltpu.VMEM((EB,), jnp.float32),
        pltpu.VMEM_SHARED((NPAD, CHUNK), jnp.float32),
        pltpu.VMEM_SHARED((NPAD,), jnp.float32),
        pltpu.SemaphoreType.DMA,
        pltpu.SemaphoreType.DMA,
        pltpu.SemaphoreType.DMA,
        pltpu.SemaphoreType.DMA,
    ],
)

_agg1nd_call = pl.kernel(
    _sc_agg1nd,
    out_type=jax.ShapeDtypeStruct((2, NPAD, CHUNK), jnp.float32),
    mesh=_MESH,
    scratch_types=[
        pltpu.VMEM((IB32, EB), jnp.int32),
        pltpu.VMEM((IB32, EB), jnp.int32),
        pltpu.VMEM((EB, CHUNK), jnp.float32),
        pltpu.VMEM((EB, CHUNK), jnp.float32),
        pltpu.VMEM_SHARED((NPAD, CHUNK), jnp.float32),
        pltpu.SemaphoreType.DMA,
        pltpu.SemaphoreType.DMA,
        pltpu.SemaphoreType.DMA,
        pltpu.SemaphoreType.DMA,
    ],
)

_agg4_call = pl.kernel(
    _sc_agg4,
    out_type=jax.ShapeDtypeStruct((4, NPAD, CHUNK), jnp.float32),
    mesh=_MESH,
    scratch_types=[
        pltpu.VMEM((IB16, EB), jnp.int32),
        pltpu.VMEM((IB16, EB), jnp.int32),
        pltpu.VMEM((EB, CHUNK), jnp.float32),
        pltpu.VMEM((EB, CHUNK), jnp.float32),
        pltpu.VMEM_SHARED((NPAD, CHUNK), jnp.float32),
        pltpu.SemaphoreType.DMA,
        pltpu.SemaphoreType.DMA,
        pltpu.SemaphoreType.DMA,
        pltpu.SemaphoreType.DMA,
    ],
)


B = 1000
NB = N // B


def _onehot_gb(batch_row):
    return (lax.broadcasted_iota(jnp.int32, (G, B), 0)
            == batch_row[None, :]).astype(jnp.float32)


def _p1l1_body(s_ref, deg_ref, x_ref, batch_ref, wl_ref, wr_ref, b_ref,
               pre_ref, stats_ref, gs, gq, gc):
    i = pl.program_id(0)
    invd = 1.0 / jnp.maximum(deg_ref[:, 0], 1.0)
    srow = (s_ref[0] + s_ref[1]) * invd[:, None]
    pre = (jnp.dot(srow, wl_ref[...], preferred_element_type=jnp.float32)
           + jnp.dot(x_ref[...], wr_ref[...], preferred_element_type=jnp.float32)
           + b_ref[...])
    pre_ref[...] = pre
    oh = _onehot_gb(batch_ref[0, 0, :])

    @pl.when(i == 0)
    def _():
        gs[...] = jnp.zeros((G, HID), jnp.float32)
        gq[...] = jnp.zeros((G, HID), jnp.float32)
        gc[...] = jnp.zeros((G, HID), jnp.float32)

    gs[...] += jnp.dot(oh, pre, preferred_element_type=jnp.float32)
    gq[...] += jnp.dot(oh, pre * pre, preferred_element_type=jnp.float32)
    gc[...] += jnp.broadcast_to(jnp.sum(oh, axis=1, keepdims=True), (G, HID))

    @pl.when(i == NB - 1)
    def _():
        stats_ref[0] = gs[...]
        stats_ref[1] = gq[...]
        stats_ref[2] = gc[...]


def _p1_body(s_ref, deg_ref, h_ref, batch_ref, wl_ref, wr_ref, b_ref,
             pre_ref, stats_ref, gs, gq, gc):
    i = pl.program_id(0)
    invd = 1.0 / jnp.maximum(deg_ref[:, 0], 1.0)
    acc = jnp.zeros((B, HID), jnp.float32)
    for c in range(4):
        sl = pl.ds(c * CHUNK, CHUNK)
        acc = acc + jnp.dot(s_ref[c] * invd[:, None], wl_ref[sl, :],
                            preferred_element_type=jnp.float32)
        acc = acc + jnp.dot(h_ref[c], wr_ref[sl, :],
                            preferred_element_type=jnp.float32)
    pre = acc + b_ref[...]
    pre_ref[...] = pre
    oh = _onehot_gb(batch_ref[0, 0, :])

    @pl.when(i == 0)
    def _():
        gs[...] = jnp.zeros((G, HID), jnp.float32)
        gq[...] = jnp.zeros((G, HID), jnp.float32)
        gc[...] = jnp.zeros((G, HID), jnp.float32)

    gs[...] += jnp.dot(oh, pre, preferred_element_type=jnp.float32)
    gq[...] += jnp.dot(oh, pre * pre, preferred_element_type=jnp.float32)
    gc[...] += jnp.broadcast_to(jnp.sum(oh, axis=1, keepdims=True), (G, HID))

    @pl.when(i == NB - 1)
    def _():
        stats_ref[0] = gs[...]
        stats_ref[1] = gq[...]
        stats_ref[2] = gc[...]


def _p2_body(pre_ref, batch_ref, stats_ref, w_ref, b_ref, ms_ref, out_ref):
    cnt = jnp.maximum(stats_ref[2], 1.0)
    mean = stats_ref[0] / cnt
    msv = ms_ref[...]
    var = stats_ref[1] / cnt - mean * mean * (2.0 * msv - msv * msv)
    rstd = lax.rsqrt(var + 1e-5)
    bt = batch_ref[0, 0, :]
    oht = (bt[:, None]
           == lax.broadcasted_iota(jnp.int32, (B, G), 1)).astype(jnp.float32)
    rmean = jnp.dot(oht, mean, preferred_element_type=jnp.float32)
    rrstd = jnp.dot(oht, rstd, preferred_element_type=jnp.float32)
    pre = pre_ref[...]
    val = (pre - rmean * msv) * rrstd * w_ref[...] + b_ref[...]
    val = jnp.maximum(val, 0.0)
    for c in range(4):
        out_ref[c] = val[:, c * CHUNK:(c + 1) * CHUNK]


def _p5a_body(h_ref, wl_ref, y_ref):
    acc = jnp.zeros((B, 128), jnp.float32)
    for c in range(4):
        sl = pl.ds(c * CHUNK, CHUNK)
        acc = acc + jnp.dot(h_ref[c], wl_ref[sl, :],
                            preferred_element_type=jnp.float32)
    y_ref[...] = acc


def _p5b_body(s_ref, deg_ref, h_ref, wr_ref, b_ref, out_ref):
    invd = 1.0 / jnp.maximum(deg_ref[:, 0], 1.0)
    acc = (s_ref[0] + s_ref[1]) * invd[:, None]
    for c in range(4):
        sl = pl.ds(c * CHUNK, CHUNK)
        acc = acc + jnp.dot(h_ref[c], wr_ref[sl, :],
                            preferred_element_type=jnp.float32)
    out_ref[...] = acc + b_ref[...]


def _p1l1_call(s2, degc, x, batch3, wlT, wrT, b2):
    return pl.pallas_call(
        _p1l1_body,
        grid=(NB,),
        in_specs=[
            pl.BlockSpec((2, B, CHUNK), lambda i: (0, i, 0)),
            pl.BlockSpec((B, 1), lambda i: (i, 0)),
            pl.BlockSpec((B, IN_F), lambda i: (i, 0)),
            pl.BlockSpec((1, 1, B), lambda i: (i, 0, 0)),
            pl.BlockSpec((IN_F, HID), lambda i: (0, 0)),
            pl.BlockSpec((IN_F, HID), lambda i: (0, 0)),
            pl.BlockSpec((1, HID), lambda i: (0, 0)),
        ],
        out_specs=[
            pl.BlockSpec((B, HID), lambda i: (i, 0)),
            pl.BlockSpec((3, G, HID), lambda i: (0, 0, 0)),
        ],
        out_shape=[jax.ShapeDtypeStruct((N, HID), jnp.float32),
                   jax.ShapeDtypeStruct((3, G, HID), jnp.float32)],
        scratch_shapes=[pltpu.VMEM((G, HID), jnp.float32)] * 3,
    )(s2, degc, x, batch3, wlT, wrT, b2)


def _p1_call(s4, degc, hc, batch3, wlT, wrT, b2):
    return pl.pallas_call(
        _p1_body,
        grid=(NB,),
        in_specs=[
            pl.BlockSpec((4, B, CHUNK), lambda i: (0, i, 0)),
            pl.BlockSpec((B, 1), lambda i: (i, 0)),
            pl.BlockSpec((4, B, CHUNK), lambda i: (0, i, 0)),
            pl.BlockSpec((1, 1, B), lambda i: (i, 0, 0)),
            pl.BlockSpec((HID, HID), lambda i: (0, 0)),
            pl.BlockSpec((HID, HID), lambda i: (0, 0)),
            pl.BlockSpec((1, HID), lambda i: (0, 0)),
        ],
        out_specs=[
            pl.BlockSpec((B, HID), lambda i: (i, 0)),
            pl.BlockSpec((3, G, HID), lambda i: (0, 0, 0)),
        ],
        out_shape=[jax.ShapeDtypeStruct((N, HID), jnp.float32),
                   jax.ShapeDtypeStruct((3, G, HID), jnp.float32)],
        scratch_shapes=[pltpu.VMEM((G, HID), jnp.float32)] * 3,
    )(s4, degc, hc, batch3, wlT, wrT, b2)


def _p2_call(pre, batch3, stats, w2, b2, ms2):
    return pl.pallas_call(
        _p2_body,
        grid=(NB,),
        in_specs=[
            pl.BlockSpec((B, HID), lambda i: (i, 0)),
            pl.BlockSpec((1, 1, B), lambda i: (i, 0, 0)),
            pl.BlockSpec((3, G, HID), lambda i: (0, 0, 0)),
            pl.BlockSpec((1, HID), lambda i: (0, 0)),
            pl.BlockSpec((1, HID), lambda i: (0, 0)),
            pl.BlockSpec((1, HID), lambda i: (0, 0)),
        ],
        out_specs=pl.BlockSpec((4, B, CHUNK), lambda i: (0, i, 0)),
        out_shape=jax.ShapeDtypeStruct((4, N, CHUNK), jnp.float32),
    )(pre, batch3, stats, w2, b2, ms2)


def _p5a_call(hc, wlT):
    return pl.pallas_call(
        _p5a_body,
        grid=(NB,),
        in_specs=[
            pl.BlockSpec((4, B, CHUNK), lambda i: (0, i, 0)),
            pl.BlockSpec((HID, 128), lambda i: (0, 0)),
        ],
        out_specs=pl.BlockSpec((B, 128), lambda i: (i, 0)),
        out_shape=jax.ShapeDtypeStruct((N, 128), jnp.float32),
    )(hc, wlT)


def _p5b_call(s2, degc, hc, wrT, b2):
    return pl.pallas_call(
        _p5b_body,
        grid=(NB,),
        in_specs=[
            pl.BlockSpec((2, B, CHUNK), lambda i: (0, i, 0)),
            pl.BlockSpec((B, 1), lambda i: (i, 0)),
            pl.BlockSpec((4, B, CHUNK), lambda i: (0, i, 0)),
            pl.BlockSpec((HID, 128), lambda i: (0, 0)),
            pl.BlockSpec((1, 128), lambda i: (0, 0)),
        ],
        out_specs=pl.BlockSpec((B, 128), lambda i: (i, 0)),
        out_shape=jax.ShapeDtypeStruct((N, 128), jnp.float32),
    )(s2, degc, hc, wrT, b2)


def kernel(x, edge_index, batch, Wl1, Wr1, b1, Wl2, Wr2, b2, Wl3, Wr3, b3,
           Wl4, Wr4, b4, Wl5, Wr5, b5, gn_w1, gn_b1, gn_ms1, gn_w2, gn_b2,
           gn_ms2, gn_w3, gn_b3, gn_ms3, gn_w4, gn_b4, gn_ms4):
    src, dst = edge_index[0], edge_index[1]
    src1, dst1 = _pad_edges(src, dst, 32, K32)
    src4, dst4 = _pad_edges(src, dst, 16, K16)
    z2 = jnp.zeros((NPAD, CHUNK), jnp.float32)
    z1 = jnp.zeros((NPAD,), jnp.float32)
    batch3 = batch.reshape(NB, 1, B)

    s2, deg = _agg1_call(x, src1, dst1, z2, z1)
    degc = (deg[0] + deg[1]).reshape(NPAD, 1)
    pre, stats = _p1l1_call(s2, degc, x, batch3, Wl1.T, Wr1.T,
                            b1.reshape(1, HID))
    h = _p2_call(pre, batch3, stats, gn_w1.reshape(1, HID),
                 gn_b1.reshape(1, HID), gn_ms1.reshape(1, HID))

    for Wl, Wr, bb, gw, gb, gms in ((Wl2, Wr2, b2, gn_w2, gn_b2, gn_ms2),
                                    (Wl3, Wr3, b3, gn_w3, gn_b3, gn_ms3),
                                    (Wl4, Wr4, b4, gn_w4, gn_b4, gn_ms4)):
        s4 = _agg4_call(h[0], h[1], h[2], h[3], src4, dst4, z2)
        pre, stats = _p1_call(s4, degc, h, batch3, Wl.T, Wr.T,
                              bb.reshape(1, HID))
        h = _p2_call(pre, batch3, stats, gw.reshape(1, HID),
                     gb.reshape(1, HID), gms.reshape(1, HID))

    wl5 = jnp.pad(Wl5.T, ((0, 0), (0, 128 - OUT_F)))
    wr5 = jnp.pad(Wr5.T, ((0, 0), (0, 128 - OUT_F)))
    b5p = jnp.pad(b5, (0, 128 - OUT_F)).reshape(1, 128)
    y = _p5a_call(h, wl5)
    s2_5 = _agg1nd_call(y, src1, dst1, z2)
    out = _p5b_call(s2_5, degc, h, wr5, b5p)
    return out[:, :OUT_F]


# speedup vs baseline: 8.1268x; 1.1356x over previous
import functools

import jax
import jax.numpy as jnp
from jax import lax
from jax.experimental import pallas as pl
from jax.experimental.pallas import tpu as pltpu
from jax.experimental.pallas import tpu_sc as plsc

N = 10000
E = 320000
IN_F = 128
HID = 512
OUT_F = 121
G = 16

NPAD = 10240
ROWS_PER_TILE = NPAD // 16
CHUNK = 128

EB = 96
NBUF = 3
IB32 = 15
NIB32 = 7
K32 = IB32 * NIB32
IB16 = 30
NIB16 = 7
K16 = IB16 * NIB16


def _pad_edges(src, dst, nshard, nib, ib):
    tot = nshard * nib * ib * EB
    pad = tot - E
    ar = jnp.arange(pad, dtype=jnp.int32)
    src_p = jnp.concatenate([src, (ar * 61) % N]).reshape(nshard, nib, ib, EB)
    dst_p = jnp.concatenate(
        [dst, N + (ar % (NPAD - N))]).reshape(nshard, nib, ib, EB)
    return src_p, dst_p


def _edge_sweep(h_hbm, src_hbm, dst_hbm, shard, acc,
                idx_s, idx_d, rows, gsems, ssems,
                ib_n, ib_sz, per_edge=None):
    nb = len(rows)

    def blk(ib, carry):
        pltpu.sync_copy(src_hbm.at[shard].at[ib], idx_s)
        pltpu.sync_copy(dst_hbm.at[shard].at[ib], idx_d)
        for q in range(nb):
            pltpu.async_copy(h_hbm.at[idx_s.at[q]], rows[q], gsems[q])

        def rot(p, carry2):
            j = nb * p
            scs = []
            for q in range(nb):
                pltpu.make_async_copy(h_hbm.at[idx_s.at[0]], rows[q],
                                      gsems[q]).wait()
                scs.append(pltpu.async_copy(rows[q], acc.at[idx_d.at[j + q]],
                                            ssems[q], add=True))
                if per_edge is not None:
                    per_edge(j + q)
            for q in range(nb):
                scs[q].wait()

                @pl.when(p + 1 < ib_sz // nb)
                def _(q=q):
                    pltpu.async_copy(h_hbm.at[idx_s.at[j + nb + q]], rows[q],
                                     gsems[q])

            return carry2

        return lax.fori_loop(0, ib_sz // nb, rot, carry)

    lax.fori_loop(0, ib_n, blk, 0)


def _sc_agg1(x_hbm, src_hbm, dst_hbm, z2_hbm, z1_hbm,
             s_out, deg_out,
             idx_s, idx_d, rows0, rows1, rows2, ones_v, acc, dacc,
             gs0, gs1, gs2, ss0, ss1, ss2):
    c = lax.axis_index("c")
    s = lax.axis_index("s")
    wid = s * 2 + c
    r0 = s * ROWS_PER_TILE
    pltpu.sync_copy(z2_hbm.at[pl.ds(r0, ROWS_PER_TILE)],
                    acc.at[pl.ds(r0, ROWS_PER_TILE)])
    pltpu.sync_copy(z1_hbm.at[pl.ds(r0, ROWS_PER_TILE)],
                    dacc.at[pl.ds(r0, ROWS_PER_TILE)])
    for j in range(EB // 16):
        ones_v[pl.ds(j * 16, 16)] = jnp.ones((16,), jnp.float32)
    plsc.subcore_barrier()

    def deg_edge(j):
        pltpu.sync_copy(ones_v, dacc.at[idx_d.at[j]], add=True)

    _edge_sweep(x_hbm, src_hbm, dst_hbm, wid, acc,
                idx_s, idx_d, [rows0, rows1, rows2],
                [gs0, gs1, gs2], [ss0, ss1, ss2],
                NIB32, IB32, per_edge=deg_edge)
    plsc.subcore_barrier()
    pltpu.sync_copy(acc.at[pl.ds(r0, ROWS_PER_TILE)],
                    s_out.at[c].at[pl.ds(r0, ROWS_PER_TILE)])
    pltpu.sync_copy(dacc.at[pl.ds(r0, ROWS_PER_TILE)],
                    deg_out.at[c].at[pl.ds(r0, ROWS_PER_TILE)])


def _sc_agg1nd(x_hbm, src_hbm, dst_hbm, z2_hbm,
               s_out,
               idx_s, idx_d, rows0, rows1, rows2, acc,
               gs0, gs1, gs2, ss0, ss1, ss2):
    c = lax.axis_index("c")
    s = lax.axis_index("s")
    wid = s * 2 + c
    r0 = s * ROWS_PER_TILE
    pltpu.sync_copy(z2_hbm.at[pl.ds(r0, ROWS_PER_TILE)],
                    acc.at[pl.ds(r0, ROWS_PER_TILE)])
    plsc.subcore_barrier()
    _edge_sweep(x_hbm, src_hbm, dst_hbm, wid, acc,
                idx_s, idx_d, [rows0, rows1, rows2],
                [gs0, gs1, gs2], [ss0, ss1, ss2],
                NIB32, IB32)
    plsc.subcore_barrier()
    pltpu.sync_copy(acc.at[pl.ds(r0, ROWS_PER_TILE)],
                    s_out.at[c].at[pl.ds(r0, ROWS_PER_TILE)])


def _sc_agg4(h0, h1, h2, h3, src_hbm, dst_hbm, z2_hbm,
             s_out,
             idx_s, idx_d, rows0, rows1, rows2, acc,
             gs0, gs1, gs2, ss0, ss1, ss2):
    c = lax.axis_index("c")
    s = lax.axis_index("s")
    r0 = s * ROWS_PER_TILE

    def do_chunk(h_hbm, out_idx):
        pltpu.sync_copy(z2_hbm.at[pl.ds(r0, ROWS_PER_TILE)],
                        acc.at[pl.ds(r0, ROWS_PER_TILE)])
        plsc.subcore_barrier()
        _edge_sweep(h_hbm, src_hbm, dst_hbm, s, acc,
                    idx_s, idx_d, [rows0, rows1, rows2],
                    [gs0, gs1, gs2], [ss0, ss1, ss2],
                    NIB16, IB16)
        plsc.subcore_barrier()
        pltpu.sync_copy(acc.at[pl.ds(r0, ROWS_PER_TILE)],
                        s_out.at[out_idx].at[pl.ds(r0, ROWS_PER_TILE)])
        plsc.subcore_barrier()

    @pl.when(c == 0)
    def _():
        do_chunk(h0, 0)
        do_chunk(h1, 1)

    @pl.when(c == 1)
    def _():
        do_chunk(h2, 2)
        do_chunk(h3, 3)


_MESH = plsc.VectorSubcoreMesh(core_axis_name="c", subcore_axis_name="s")

_agg1_call = pl.kernel(
    _sc_agg1,
    out_type=[jax.ShapeDtypeStruct((2, NPAD, CHUNK), jnp.float32),
              jax.ShapeDtypeStruct((2, NPAD), jnp.float32)],
    mesh=_MESH,
    scratch_types=[
        pltpu.VMEM((IB32, EB), jnp.int32),
        pltpu.VMEM((IB32, EB), jnp.int32),
    ] + [pltpu.VMEM((EB, CHUNK), jnp.float32)] * NBUF + [
        pltpu.VMEM((EB,), jnp.float32),
        pltpu.VMEM_SHARED((NPAD, CHUNK), jnp.float32),
        pltpu.VMEM_SHARED((NPAD,), jnp.float32),
    ] + [pltpu.SemaphoreType.DMA] * (2 * NBUF),
)

_agg1nd_call = pl.kernel(
    _sc_agg1nd,
    out_type=jax.ShapeDtypeStruct((2, NPAD, CHUNK), jnp.float32),
    mesh=_MESH,
    scratch_types=[
        pltpu.VMEM((IB32, EB), jnp.int32),
        pltpu.VMEM((IB32, EB), jnp.int32),
    ] + [pltpu.VMEM((EB, CHUNK), jnp.float32)] * NBUF + [
        pltpu.VMEM_SHARED((NPAD, CHUNK), jnp.float32),
    ] + [pltpu.SemaphoreType.DMA] * (2 * NBUF),
)

_agg4_call = pl.kernel(
    _sc_agg4,
    out_type=jax.ShapeDtypeStruct((4, NPAD, CHUNK), jnp.float32),
    mesh=_MESH,
    scratch_types=[
        pltpu.VMEM((IB16, EB), jnp.int32),
        pltpu.VMEM((IB16, EB), jnp.int32),
    ] + [pltpu.VMEM((EB, CHUNK), jnp.float32)] * NBUF + [
        pltpu.VMEM_SHARED((NPAD, CHUNK), jnp.float32),
    ] + [pltpu.SemaphoreType.DMA] * (2 * NBUF),
)


B = 1000
NB = N // B


def _onehot_gb(batch_row):
    return (lax.broadcasted_iota(jnp.int32, (G, B), 0)
            == batch_row[None, :]).astype(jnp.float32)


def _p1l1_body(s_ref, deg_ref, x_ref, batch_ref, wl_ref, wr_ref, b_ref,
               pre_ref, stats_ref, gs, gq, gc):
    i = pl.program_id(0)
    invd = 1.0 / jnp.maximum(deg_ref[:, 0], 1.0)
    srow = (s_ref[0] + s_ref[1]) * invd[:, None]
    pre = (jnp.dot(srow, wl_ref[...], preferred_element_type=jnp.float32)
           + jnp.dot(x_ref[...], wr_ref[...], preferred_element_type=jnp.float32)
           + b_ref[...])
    pre_ref[...] = pre
    oh = _onehot_gb(batch_ref[0, 0, :])

    @pl.when(i == 0)
    def _():
        gs[...] = jnp.zeros((G, HID), jnp.float32)
        gq[...] = jnp.zeros((G, HID), jnp.float32)
        gc[...] = jnp.zeros((G, HID), jnp.float32)

    gs[...] += jnp.dot(oh, pre, preferred_element_type=jnp.float32)
    gq[...] += jnp.dot(oh, pre * pre, preferred_element_type=jnp.float32)
    gc[...] += jnp.broadcast_to(jnp.sum(oh, axis=1, keepdims=True), (G, HID))

    @pl.when(i == NB - 1)
    def _():
        stats_ref[0] = gs[...]
        stats_ref[1] = gq[...]
        stats_ref[2] = gc[...]


def _p1_body(s_ref, deg_ref, h_ref, batch_ref, wl_ref, wr_ref, b_ref,
             pre_ref, stats_ref, gs, gq, gc):
    i = pl.program_id(0)
    invd = 1.0 / jnp.maximum(deg_ref[:, 0], 1.0)
    acc = jnp.zeros((B, HID), jnp.float32)
    for c in range(4):
        sl = pl.ds(c * CHUNK, CHUNK)
        acc = acc + jnp.dot(s_ref[c] * invd[:, None], wl_ref[sl, :],
                            preferred_element_type=jnp.float32)
        acc = acc + jnp.dot(h_ref[c], wr_ref[sl, :],
                            preferred_element_type=jnp.float32)
    pre = acc + b_ref[...]
    pre_ref[...] = pre
    oh = _onehot_gb(batch_ref[0, 0, :])

    @pl.when(i == 0)
    def _():
        gs[...] = jnp.zeros((G, HID), jnp.float32)
        gq[...] = jnp.zeros((G, HID), jnp.float32)
        gc[...] = jnp.zeros((G, HID), jnp.float32)

    gs[...] += jnp.dot(oh, pre, preferred_element_type=jnp.float32)
    gq[...] += jnp.dot(oh, pre * pre, preferred_element_type=jnp.float32)
    gc[...] += jnp.broadcast_to(jnp.sum(oh, axis=1, keepdims=True), (G, HID))

    @pl.when(i == NB - 1)
    def _():
        stats_ref[0] = gs[...]
        stats_ref[1] = gq[...]
        stats_ref[2] = gc[...]


def _p2_body(pre_ref, batch_ref, stats_ref, w_ref, b_ref, ms_ref, out_ref):
    cnt = jnp.maximum(stats_ref[2], 1.0)
    mean = stats_ref[0] / cnt
    msv = ms_ref[...]
    var = stats_ref[1] / cnt - mean * mean * (2.0 * msv - msv * msv)
    rstd = lax.rsqrt(var + 1e-5)
    bt = batch_ref[0, 0, :]
    oht = (bt[:, None]
           == lax.broadcasted_iota(jnp.int32, (B, G), 1)).astype(jnp.float32)
    rmean = jnp.dot(oht, mean, preferred_element_type=jnp.float32)
    rrstd = jnp.dot(oht, rstd, preferred_element_type=jnp.float32)
    pre = pre_ref[...]
    val = (pre - rmean * msv) * rrstd * w_ref[...] + b_ref[...]
    val = jnp.maximum(val, 0.0)
    for c in range(4):
        out_ref[c] = val[:, c * CHUNK:(c + 1) * CHUNK]


def _p5a_body(h_ref, wl_ref, y_ref):
    acc = jnp.zeros((B, 128), jnp.float32)
    for c in range(4):
        sl = pl.ds(c * CHUNK, CHUNK)
        acc = acc + jnp.dot(h_ref[c], wl_ref[sl, :],
                            preferred_element_type=jnp.float32)
    y_ref[...] = acc


def _p5b_body(s_ref, deg_ref, h_ref, wr_ref, b_ref, out_ref):
    invd = 1.0 / jnp.maximum(deg_ref[:, 0], 1.0)
    acc = (s_ref[0] + s_ref[1]) * invd[:, None]
    for c in range(4):
        sl = pl.ds(c * CHUNK, CHUNK)
        acc = acc + jnp.dot(h_ref[c], wr_ref[sl, :],
                            preferred_element_type=jnp.float32)
    out_ref[...] = acc + b_ref[...]


def _p1l1_call(s2, degc, x, batch3, wlT, wrT, b2):
    return pl.pallas_call(
        _p1l1_body,
        grid=(NB,),
        in_specs=[
            pl.BlockSpec((2, B, CHUNK), lambda i: (0, i, 0)),
            pl.BlockSpec((B, 1), lambda i: (i, 0)),
            pl.BlockSpec((B, IN_F), lambda i: (i, 0)),
            pl.BlockSpec((1, 1, B), lambda i: (i, 0, 0)),
            pl.BlockSpec((IN_F, HID), lambda i: (0, 0)),
            pl.BlockSpec((IN_F, HID), lambda i: (0, 0)),
            pl.BlockSpec((1, HID), lambda i: (0, 0)),
        ],
        out_specs=[
            pl.BlockSpec((B, HID), lambda i: (i, 0)),
            pl.BlockSpec((3, G, HID), lambda i: (0, 0, 0)),
        ],
        out_shape=[jax.ShapeDtypeStruct((N, HID), jnp.float32),
                   jax.ShapeDtypeStruct((3, G, HID), jnp.float32)],
        scratch_shapes=[pltpu.VMEM((G, HID), jnp.float32)] * 3,
    )(s2, degc, x, batch3, wlT, wrT, b2)


def _p1_call(s4, degc, hc, batch3, wlT, wrT, b2):
    return pl.pallas_call(
        _p1_body,
        grid=(NB,),
        in_specs=[
            pl.BlockSpec((4, B, CHUNK), lambda i: (0, i, 0)),
            pl.BlockSpec((B, 1), lambda i: (i, 0)),
            pl.BlockSpec((4, B, CHUNK), lambda i: (0, i, 0)),
            pl.BlockSpec((1, 1, B), lambda i: (i, 0, 0)),
            pl.BlockSpec((HID, HID), lambda i: (0, 0)),
            pl.BlockSpec((HID, HID), lambda i: (0, 0)),
            pl.BlockSpec((1, HID), lambda i: (0, 0)),
        ],
        out_specs=[
            pl.BlockSpec((B, HID), lambda i: (i, 0)),
            pl.BlockSpec((3, G, HID), lambda i: (0, 0, 0)),
        ],
        out_shape=[jax.ShapeDtypeStruct((N, HID), jnp.float32),
                   jax.ShapeDtypeStruct((3, G, HID), jnp.float32)],
        scratch_shapes=[pltpu.VMEM((G, HID), jnp.float32)] * 3,
    )(s4, degc, hc, batch3, wlT, wrT, b2)


def _p2_call(pre, batch3, stats, w2, b2, ms2):
    return pl.pallas_call(
        _p2_body,
        grid=(NB,),
        in_specs=[
            pl.BlockSpec((B, HID), lambda i: (i, 0)),
            pl.BlockSpec((1, 1, B), lambda i: (i, 0, 0)),
            pl.BlockSpec((3, G, HID), lambda i: (0, 0, 0)),
            pl.BlockSpec((1, HID), lambda i: (0, 0)),
            pl.BlockSpec((1, HID), lambda i: (0, 0)),
            pl.BlockSpec((1, HID), lambda i: (0, 0)),
        ],
        out_specs=pl.BlockSpec((4, B, CHUNK), lambda i: (0, i, 0)),
        out_shape=jax.ShapeDtypeStruct((4, N, CHUNK), jnp.float32),
    )(pre, batch3, stats, w2, b2, ms2)


def _p5a_call(hc, wlT):
    return pl.pallas_call(
        _p5a_body,
        grid=(NB,),
        in_specs=[
            pl.BlockSpec((4, B, CHUNK), lambda i: (0, i, 0)),
            pl.BlockSpec((HID, 128), lambda i: (0, 0)),
        ],
        out_specs=pl.BlockSpec((B, 128), lambda i: (i, 0)),
        out_shape=jax.ShapeDtypeStruct((N, 128), jnp.float32),
    )(hc, wlT)


def _p5b_call(s2, degc, hc, wrT, b2):
    return pl.pallas_call(
        _p5b_body,
        grid=(NB,),
        in_specs=[
            pl.BlockSpec((2, B, CHUNK), lambda i: (0, i, 0)),
            pl.BlockSpec((B, 1), lambda i: (i, 0)),
            pl.BlockSpec((4, B, CHUNK), lambda i: (0, i, 0)),
            pl.BlockSpec((HID, 128), lambda i: (0, 0)),
            pl.BlockSpec((1, 128), lambda i: (0, 0)),
        ],
        out_specs=pl.BlockSpec((B, 128), lambda i: (i, 0)),
        out_shape=jax.ShapeDtypeStruct((N, 128), jnp.float32),
    )(s2, degc, hc, wrT, b2)


def kernel(x, edge_index, batch, Wl1, Wr1, b1, Wl2, Wr2, b2, Wl3, Wr3, b3,
           Wl4, Wr4, b4, Wl5, Wr5, b5, gn_w1, gn_b1, gn_ms1, gn_w2, gn_b2,
           gn_ms2, gn_w3, gn_b3, gn_ms3, gn_w4, gn_b4, gn_ms4):
    src, dst = edge_index[0], edge_index[1]
    src1, dst1 = _pad_edges(src, dst, 32, NIB32, IB32)
    src4, dst4 = _pad_edges(src, dst, 16, NIB16, IB16)
    z2 = jnp.zeros((NPAD, CHUNK), jnp.float32)
    z1 = jnp.zeros((NPAD,), jnp.float32)
    batch3 = batch.reshape(NB, 1, B)

    s2, deg = _agg1_call(x, src1, dst1, z2, z1)
    degc = (deg[0] + deg[1]).reshape(NPAD, 1)
    pre, stats = _p1l1_call(s2, degc, x, batch3, Wl1.T, Wr1.T,
                            b1.reshape(1, HID))
    h = _p2_call(pre, batch3, stats, gn_w1.reshape(1, HID),
                 gn_b1.reshape(1, HID), gn_ms1.reshape(1, HID))

    for Wl, Wr, bb, gw, gb, gms in ((Wl2, Wr2, b2, gn_w2, gn_b2, gn_ms2),
                                    (Wl3, Wr3, b3, gn_w3, gn_b3, gn_ms3),
                                    (Wl4, Wr4, b4, gn_w4, gn_b4, gn_ms4)):
        s4 = _agg4_call(h[0], h[1], h[2], h[3], src4, dst4, z2)
        pre, stats = _p1_call(s4, degc, h, batch3, Wl.T, Wr.T,
                              bb.reshape(1, HID))
        h = _p2_call(pre, batch3, stats, gw.reshape(1, HID),
                     gb.reshape(1, HID), gms.reshape(1, HID))

    wl5 = jnp.pad(Wl5.T, ((0, 0), (0, 128 - OUT_F)))
    wr5 = jnp.pad(Wr5.T, ((0, 0), (0, 128 - OUT_F)))
    b5p = jnp.pad(b5, (0, 128 - OUT_F)).reshape(1, 128)
    y = _p5a_call(h, wl5)
    s2_5 = _agg1nd_call(y, src1, dst1, z2)
    out = _p5b_call(s2_5, degc, h, wr5, b5p)
    return out[:, :OUT_F]


# speedup vs baseline: 8.1546x; 1.0034x over previous
import functools

import jax
import jax.numpy as jnp
from jax import lax
from jax.experimental import pallas as pl
from jax.experimental.pallas import tpu as pltpu
from jax.experimental.pallas import tpu_sc as plsc

N = 10000
E = 320000
IN_F = 128
HID = 512
OUT_F = 121
G = 16

NPAD = 10240
ROWS_PER_TILE = NPAD // 16
CHUNK = 128

EB = 96
NBUF = 3
IB32 = 15
NIB32 = 7
K32 = IB32 * NIB32
IB16 = 30
NIB16 = 7
K16 = IB16 * NIB16


def _pad_edges(src, dst, nshard, nib, ib):
    tot = nshard * nib * ib * EB
    pad = tot - E
    ar = jnp.arange(pad, dtype=jnp.int32)
    src_p = jnp.concatenate([src, (ar * 61) % N]).reshape(nshard, nib, ib, EB)
    dst_p = jnp.concatenate(
        [dst, N + (ar % (NPAD - N))]).reshape(nshard, nib, ib, EB)
    return src_p, dst_p


def _edge_sweep(h_hbm, src_hbm, dst_hbm, shard, acc,
                idx_s, idx_d, rows, gsems, ssems,
                ib_n, ib_sz, per_edge=None):
    nb = len(rows)

    def blk(ib, carry):
        pltpu.sync_copy(src_hbm.at[shard].at[ib], idx_s)
        pltpu.sync_copy(dst_hbm.at[shard].at[ib], idx_d)
        for q in range(nb):
            pltpu.async_copy(h_hbm.at[idx_s.at[q]], rows[q], gsems[q])

        def rot(p, carry2):
            j = nb * p
            scs = []
            for q in range(nb):
                pltpu.make_async_copy(h_hbm.at[idx_s.at[0]], rows[q],
                                      gsems[q]).wait()
                scs.append(pltpu.async_copy(rows[q], acc.at[idx_d.at[j + q]],
                                            ssems[q], add=True))
                if per_edge is not None:
                    per_edge(j + q)
            for q in range(nb):
                scs[q].wait()

                @pl.when(p + 1 < ib_sz // nb)
                def _(q=q):
                    pltpu.async_copy(h_hbm.at[idx_s.at[j + nb + q]], rows[q],
                                     gsems[q])

            return carry2

        return lax.fori_loop(0, ib_sz // nb, rot, carry)

    lax.fori_loop(0, ib_n, blk, 0)


def _sc_agg1(x_hbm, src_hbm, dst_hbm, z2_hbm, z1_hbm,
             s_out, deg_out,
             idx_s, idx_d, rows0, rows1, rows2, ones_v, acc, dacc,
             gs0, gs1, gs2, ss0, ss1, ss2):
    c = lax.axis_index("c")
    s = lax.axis_index("s")
    wid = s * 2 + c
    r0 = s * ROWS_PER_TILE
    pltpu.sync_copy(z2_hbm.at[pl.ds(r0, ROWS_PER_TILE)],
                    acc.at[pl.ds(r0, ROWS_PER_TILE)])
    pltpu.sync_copy(z1_hbm.at[pl.ds(r0, ROWS_PER_TILE)],
                    dacc.at[pl.ds(r0, ROWS_PER_TILE)])
    for j in range(EB // 16):
        ones_v[pl.ds(j * 16, 16)] = jnp.ones((16,), jnp.float32)
    plsc.subcore_barrier()

    def deg_edge(j):
        pltpu.sync_copy(ones_v, dacc.at[idx_d.at[j]], add=True)

    _edge_sweep(x_hbm, src_hbm, dst_hbm, wid, acc,
                idx_s, idx_d, [rows0, rows1, rows2],
                [gs0, gs1, gs2], [ss0, ss1, ss2],
                NIB32, IB32, per_edge=deg_edge)
    plsc.subcore_barrier()
    pltpu.sync_copy(acc.at[pl.ds(r0, ROWS_PER_TILE)],
                    s_out.at[c].at[pl.ds(r0, ROWS_PER_TILE)])
    pltpu.sync_copy(dacc.at[pl.ds(r0, ROWS_PER_TILE)],
                    deg_out.at[c].at[pl.ds(r0, ROWS_PER_TILE)])


def _sc_agg1nd(x_hbm, src_hbm, dst_hbm, z2_hbm,
               s_out,
               idx_s, idx_d, rows0, rows1, rows2, acc,
               gs0, gs1, gs2, ss0, ss1, ss2):
    c = lax.axis_index("c")
    s = lax.axis_index("s")
    wid = s * 2 + c
    r0 = s * ROWS_PER_TILE
    pltpu.sync_copy(z2_hbm.at[pl.ds(r0, ROWS_PER_TILE)],
                    acc.at[pl.ds(r0, ROWS_PER_TILE)])
    plsc.subcore_barrier()
    _edge_sweep(x_hbm, src_hbm, dst_hbm, wid, acc,
                idx_s, idx_d, [rows0, rows1, rows2],
                [gs0, gs1, gs2], [ss0, ss1, ss2],
                NIB32, IB32)
    plsc.subcore_barrier()
    pltpu.sync_copy(acc.at[pl.ds(r0, ROWS_PER_TILE)],
                    s_out.at[c].at[pl.ds(r0, ROWS_PER_TILE)])


def _sc_agg4(h0, h1, h2, h3, src_hbm, dst_hbm, z2_hbm,
             s_out,
             idx_s, idx_d, rows0, rows1, rows2, acc,
             gs0, gs1, gs2, ss0, ss1, ss2):
    c = lax.axis_index("c")
    s = lax.axis_index("s")
    r0 = s * ROWS_PER_TILE

    def do_chunk(h_hbm, out_idx):
        pltpu.sync_copy(z2_hbm.at[pl.ds(r0, ROWS_PER_TILE)],
                        acc.at[pl.ds(r0, ROWS_PER_TILE)])
        plsc.subcore_barrier()
        _edge_sweep(h_hbm, src_hbm, dst_hbm, s, acc,
                    idx_s, idx_d, [rows0, rows1, rows2],
                    [gs0, gs1, gs2], [ss0, ss1, ss2],
                    NIB16, IB16)
        plsc.subcore_barrier()
        pltpu.sync_copy(acc.at[pl.ds(r0, ROWS_PER_TILE)],
                        s_out.at[out_idx].at[pl.ds(r0, ROWS_PER_TILE)])
        plsc.subcore_barrier()

    @pl.when(c == 0)
    def _():
        do_chunk(h0, 0)
        do_chunk(h1, 1)

    @pl.when(c == 1)
    def _():
        do_chunk(h2, 2)
        do_chunk(h3, 3)


_MESH = plsc.VectorSubcoreMesh(core_axis_name="c", subcore_axis_name="s")

_agg1_call = pl.kernel(
    _sc_agg1,
    out_type=[jax.ShapeDtypeStruct((2, NPAD, CHUNK), jnp.float32),
              jax.ShapeDtypeStruct((2, NPAD), jnp.float32)],
    mesh=_MESH,
    scratch_types=[
        pltpu.VMEM((IB32, EB), jnp.int32),
        pltpu.VMEM((IB32, EB), jnp.int32),
    ] + [pltpu.VMEM((EB, CHUNK), jnp.float32)] * NBUF + [
        pltpu.VMEM((EB,), jnp.float32),
        pltpu.VMEM_SHARED((NPAD, CHUNK), jnp.float32),
        pltpu.VMEM_SHARED((NPAD,), jnp.float32),
    ] + [pltpu.SemaphoreType.DMA] * (2 * NBUF),
)

_agg1nd_call = pl.kernel(
    _sc_agg1nd,
    out_type=jax.ShapeDtypeStruct((2, NPAD, CHUNK), jnp.float32),
    mesh=_MESH,
    scratch_types=[
        pltpu.VMEM((IB32, EB), jnp.int32),
        pltpu.VMEM((IB32, EB), jnp.int32),
    ] + [pltpu.VMEM((EB, CHUNK), jnp.float32)] * NBUF + [
        pltpu.VMEM_SHARED((NPAD, CHUNK), jnp.float32),
    ] + [pltpu.SemaphoreType.DMA] * (2 * NBUF),
)

_agg4_call = pl.kernel(
    _sc_agg4,
    out_type=jax.ShapeDtypeStruct((4, NPAD, CHUNK), jnp.float32),
    mesh=_MESH,
    scratch_types=[
        pltpu.VMEM((IB16, EB), jnp.int32),
        pltpu.VMEM((IB16, EB), jnp.int32),
    ] + [pltpu.VMEM((EB, CHUNK), jnp.float32)] * NBUF + [
        pltpu.VMEM_SHARED((NPAD, CHUNK), jnp.float32),
    ] + [pltpu.SemaphoreType.DMA] * (2 * NBUF),
)


B = 1000
NB = N // B


def _onehot_gb(batch_row):
    return (lax.broadcasted_iota(jnp.int32, (G, B), 0)
            == batch_row[None, :]).astype(jnp.float32)


def _p1a1_body(x_ref, wr_ref, b_ref, r_ref):
    r_ref[...] = (jnp.dot(x_ref[...], wr_ref[...],
                          preferred_element_type=jnp.float32) + b_ref[...])


def _p1a_body(h_ref, wr_ref, b_ref, r_ref):
    acc = jnp.zeros((B, HID), jnp.float32)
    for c in range(4):
        sl = pl.ds(c * CHUNK, CHUNK)
        acc = acc + jnp.dot(h_ref[c], wr_ref[sl, :],
                            preferred_element_type=jnp.float32)
    r_ref[...] = acc + b_ref[...]


def _p1b1_body(s_ref, deg_ref, r_ref, batch_ref, wl_ref,
               pre_ref, stats_ref, gs, gq, gc):
    i = pl.program_id(0)
    invd = 1.0 / jnp.maximum(deg_ref[:, 0], 1.0)
    srow = (s_ref[0] + s_ref[1]) * invd[:, None]
    pre = (jnp.dot(srow, wl_ref[...], preferred_element_type=jnp.float32)
           + r_ref[...])
    pre_ref[...] = pre
    oh = _onehot_gb(batch_ref[0, 0, :])

    @pl.when(i == 0)
    def _():
        gs[...] = jnp.zeros((G, HID), jnp.float32)
        gq[...] = jnp.zeros((G, HID), jnp.float32)
        gc[...] = jnp.zeros((G, HID), jnp.float32)

    gs[...] += jnp.dot(oh, pre, preferred_element_type=jnp.float32)
    gq[...] += jnp.dot(oh, pre * pre, preferred_element_type=jnp.float32)
    gc[...] += jnp.broadcast_to(jnp.sum(oh, axis=1, keepdims=True), (G, HID))

    @pl.when(i == NB - 1)
    def _():
        stats_ref[0] = gs[...]
        stats_ref[1] = gq[...]
        stats_ref[2] = gc[...]


def _p1b_body(s_ref, deg_ref, r_ref, batch_ref, wl_ref,
              pre_ref, stats_ref, gs, gq, gc):
    i = pl.program_id(0)
    invd = 1.0 / jnp.maximum(deg_ref[:, 0], 1.0)
    acc = r_ref[...]
    for c in range(4):
        sl = pl.ds(c * CHUNK, CHUNK)
        acc = acc + jnp.dot(s_ref[c] * invd[:, None], wl_ref[sl, :],
                            preferred_element_type=jnp.float32)
    pre = acc
    pre_ref[...] = pre
    oh = _onehot_gb(batch_ref[0, 0, :])

    @pl.when(i == 0)
    def _():
        gs[...] = jnp.zeros((G, HID), jnp.float32)
        gq[...] = jnp.zeros((G, HID), jnp.float32)
        gc[...] = jnp.zeros((G, HID), jnp.float32)

    gs[...] += jnp.dot(oh, pre, preferred_element_type=jnp.float32)
    gq[...] += jnp.dot(oh, pre * pre, preferred_element_type=jnp.float32)
    gc[...] += jnp.broadcast_to(jnp.sum(oh, axis=1, keepdims=True), (G, HID))

    @pl.when(i == NB - 1)
    def _():
        stats_ref[0] = gs[...]
        stats_ref[1] = gq[...]
        stats_ref[2] = gc[...]


def _p2_body(pre_ref, batch_ref, stats_ref, w_ref, b_ref, ms_ref, out_ref):
    cnt = jnp.maximum(stats_ref[2], 1.0)
    mean = stats_ref[0] / cnt
    msv = ms_ref[...]
    var = stats_ref[1] / cnt - mean * mean * (2.0 * msv - msv * msv)
    rstd = lax.rsqrt(var + 1e-5)
    bt = batch_ref[0, 0, :]
    oht = (bt[:, None]
           == lax.broadcasted_iota(jnp.int32, (B, G), 1)).astype(jnp.float32)
    rmean = jnp.dot(oht, mean, preferred_element_type=jnp.float32)
    rrstd = jnp.dot(oht, rstd, preferred_element_type=jnp.float32)
    pre = pre_ref[...]
    val = (pre - rmean * msv) * rrstd * w_ref[...] + b_ref[...]
    val = jnp.maximum(val, 0.0)
    for c in range(4):
        out_ref[c] = val[:, c * CHUNK:(c + 1) * CHUNK]


def _p5a_body(h_ref, wl_ref, y_ref):
    acc = jnp.zeros((B, 128), jnp.float32)
    for c in range(4):
        sl = pl.ds(c * CHUNK, CHUNK)
        acc = acc + jnp.dot(h_ref[c], wl_ref[sl, :],
                            preferred_element_type=jnp.float32)
    y_ref[...] = acc


def _p5c_body(s_ref, deg_ref, r_ref, out_ref):
    invd = 1.0 / jnp.maximum(deg_ref[:, 0], 1.0)
    out_ref[...] = (s_ref[0] + s_ref[1]) * invd[:, None] + r_ref[...]


def _p1a1_call(x, wrT, b2):
    return pl.pallas_call(
        _p1a1_body,
        grid=(NB,),
        in_specs=[
            pl.BlockSpec((B, IN_F), lambda i: (i, 0)),
            pl.BlockSpec((IN_F, HID), lambda i: (0, 0)),
            pl.BlockSpec((1, HID), lambda i: (0, 0)),
        ],
        out_specs=pl.BlockSpec((B, HID), lambda i: (i, 0)),
        out_shape=jax.ShapeDtypeStruct((N, HID), jnp.float32),
    )(x, wrT, b2)


def _p1a_call(hc, wrT, b2):
    return pl.pallas_call(
        _p1a_body,
        grid=(NB,),
        in_specs=[
            pl.BlockSpec((4, B, CHUNK), lambda i: (0, i, 0)),
            pl.BlockSpec((HID, HID), lambda i: (0, 0)),
            pl.BlockSpec((1, HID), lambda i: (0, 0)),
        ],
        out_specs=pl.BlockSpec((B, HID), lambda i: (i, 0)),
        out_shape=jax.ShapeDtypeStruct((N, HID), jnp.float32),
    )(hc, wrT, b2)


def _p1b1_call(s2, degc, r, batch3, wlT):
    return pl.pallas_call(
        _p1b1_body,
        grid=(NB,),
        in_specs=[
            pl.BlockSpec((2, B, CHUNK), lambda i: (0, i, 0)),
            pl.BlockSpec((B, 1), lambda i: (i, 0)),
            pl.BlockSpec((B, HID), lambda i: (i, 0)),
            pl.BlockSpec((1, 1, B), lambda i: (i, 0, 0)),
            pl.BlockSpec((IN_F, HID), lambda i: (0, 0)),
        ],
        out_specs=[
            pl.BlockSpec((B, HID), lambda i: (i, 0)),
            pl.BlockSpec((3, G, HID), lambda i: (0, 0, 0)),
        ],
        out_shape=[jax.ShapeDtypeStruct((N, HID), jnp.float32),
                   jax.ShapeDtypeStruct((3, G, HID), jnp.float32)],
        scratch_shapes=[pltpu.VMEM((G, HID), jnp.float32)] * 3,
    )(s2, degc, r, batch3, wlT)


def _p1b_call(s4, degc, r, batch3, wlT):
    return pl.pallas_call(
        _p1b_body,
        grid=(NB,),
        in_specs=[
            pl.BlockSpec((4, B, CHUNK), lambda i: (0, i, 0)),
            pl.BlockSpec((B, 1), lambda i: (i, 0)),
            pl.BlockSpec((B, HID), lambda i: (i, 0)),
            pl.BlockSpec((1, 1, B), lambda i: (i, 0, 0)),
            pl.BlockSpec((HID, HID), lambda i: (0, 0)),
        ],
        out_specs=[
            pl.BlockSpec((B, HID), lambda i: (i, 0)),
            pl.BlockSpec((3, G, HID), lambda i: (0, 0, 0)),
        ],
        out_shape=[jax.ShapeDtypeStruct((N, HID), jnp.float32),
                   jax.ShapeDtypeStruct((3, G, HID), jnp.float32)],
        scratch_shapes=[pltpu.VMEM((G, HID), jnp.float32)] * 3,
    )(s4, degc, r, batch3, wlT)


def _p2_call(pre, batch3, stats, w2, b2, ms2):
    return pl.pallas_call(
        _p2_body,
        grid=(NB,),
        in_specs=[
            pl.BlockSpec((B, HID), lambda i: (i, 0)),
            pl.BlockSpec((1, 1, B), lambda i: (i, 0, 0)),
            pl.BlockSpec((3, G, HID), lambda i: (0, 0, 0)),
            pl.BlockSpec((1, HID), lambda i: (0, 0)),
            pl.BlockSpec((1, HID), lambda i: (0, 0)),
            pl.BlockSpec((1, HID), lambda i: (0, 0)),
        ],
        out_specs=pl.BlockSpec((4, B, CHUNK), lambda i: (0, i, 0)),
        out_shape=jax.ShapeDtypeStruct((4, N, CHUNK), jnp.float32),
    )(pre, batch3, stats, w2, b2, ms2)


def _p5a_call(hc, wlT):
    return pl.pallas_call(
        _p5a_body,
        grid=(NB,),
        in_specs=[
            pl.BlockSpec((4, B, CHUNK), lambda i: (0, i, 0)),
            pl.BlockSpec((HID, 128), lambda i: (0, 0)),
        ],
        out_specs=pl.BlockSpec((B, 128), lambda i: (i, 0)),
        out_shape=jax.ShapeDtypeStruct((N, 128), jnp.float32),
    )(hc, wlT)


def _p5r_call(hc, wrT, b2):
    def body(h_ref, wr_ref, b_ref, r_ref):
        acc = jnp.zeros((B, 128), jnp.float32)
        for c in range(4):
            sl = pl.ds(c * CHUNK, CHUNK)
            acc = acc + jnp.dot(h_ref[c], wr_ref[sl, :],
                                preferred_element_type=jnp.float32)
        r_ref[...] = acc + b_ref[...]

    return pl.pallas_call(
        body,
        grid=(NB,),
        in_specs=[
            pl.BlockSpec((4, B, CHUNK), lambda i: (0, i, 0)),
            pl.BlockSpec((HID, 128), lambda i: (0, 0)),
            pl.BlockSpec((1, 128), lambda i: (0, 0)),
        ],
        out_specs=pl.BlockSpec((B, 128), lambda i: (i, 0)),
        out_shape=jax.ShapeDtypeStruct((N, 128), jnp.float32),
    )(hc, wrT, b2)


def _p5c_call(s2, degc, r5):
    return pl.pallas_call(
        _p5c_body,
        grid=(NB,),
        in_specs=[
            pl.BlockSpec((2, B, CHUNK), lambda i: (0, i, 0)),
            pl.BlockSpec((B, 1), lambda i: (i, 0)),
            pl.BlockSpec((B, 128), lambda i: (i, 0)),
        ],
        out_specs=pl.BlockSpec((B, 128), lambda i: (i, 0)),
        out_shape=jax.ShapeDtypeStruct((N, 128), jnp.float32),
    )(s2, degc, r5)


def kernel(x, edge_index, batch, Wl1, Wr1, b1, Wl2, Wr2, b2, Wl3, Wr3, b3,
           Wl4, Wr4, b4, Wl5, Wr5, b5, gn_w1, gn_b1, gn_ms1, gn_w2, gn_b2,
           gn_ms2, gn_w3, gn_b3, gn_ms3, gn_w4, gn_b4, gn_ms4):
    src, dst = edge_index[0], edge_index[1]
    src1, dst1 = _pad_edges(src, dst, 32, NIB32, IB32)
    src4, dst4 = _pad_edges(src, dst, 16, NIB16, IB16)
    z2 = jnp.zeros((NPAD, CHUNK), jnp.float32)
    z1 = jnp.zeros((NPAD,), jnp.float32)
    batch3 = batch.reshape(NB, 1, B)

    s2, deg = _agg1_call(x, src1, dst1, z2, z1)
    r = _p1a1_call(x, Wr1.T, b1.reshape(1, HID))
    degc = (deg[0] + deg[1]).reshape(NPAD, 1)
    pre, stats = _p1b1_call(s2, degc, r, batch3, Wl1.T)
    h = _p2_call(pre, batch3, stats, gn_w1.reshape(1, HID),
                 gn_b1.reshape(1, HID), gn_ms1.reshape(1, HID))

    for Wl, Wr, bb, gw, gb, gms in ((Wl2, Wr2, b2, gn_w2, gn_b2, gn_ms2),
                                    (Wl3, Wr3, b3, gn_w3, gn_b3, gn_ms3),
                                    (Wl4, Wr4, b4, gn_w4, gn_b4, gn_ms4)):
        s4 = _agg4_call(h[0], h[1], h[2], h[3], src4, dst4, z2)
        r = _p1a_call(h, Wr.T, bb.reshape(1, HID))
        pre, stats = _p1b_call(s4, degc, r, batch3, Wl.T)
        h = _p2_call(pre, batch3, stats, gw.reshape(1, HID),
                     gb.reshape(1, HID), gms.reshape(1, HID))

    wl5 = jnp.pad(Wl5.T, ((0, 0), (0, 128 - OUT_F)))
    wr5 = jnp.pad(Wr5.T, ((0, 0), (0, 128 - OUT_F)))
    b5p = jnp.pad(b5, (0, 128 - OUT_F)).reshape(1, 128)
    y = _p5a_call(h, wl5)
    s2_5 = _agg1nd_call(y, src1, dst1, z2)
    r5 = _p5r_call(h, wr5, b5p)
    out = _p5c_call(s2_5, degc, r5)
    return out[:, :OUT_F]
